# baseline (speedup 1.0000x reference)
import functools

import jax
import jax.numpy as jnp
from jax import lax
from jax.experimental import pallas as pl
from jax.experimental.pallas import tpu as pltpu
from jax.experimental.pallas import tpu_sc as plsc

EMBED = 64
OUT = 64
LANES = 16

_NC = 2
_NS = 16
_NW = _NC * _NS

_B = 4096
_L = 200
_BSLAB = _B // _NW
_CL = 4
_NCHUNKS = _L // _CL
_TILE = 8 * 128


def _tables_body(wW_ref, wK_ref, fcW_ref, fcb_ref, tW_ref, tK_ref):
    fw = fcW_ref[...]
    tW_ref[...] = jnp.dot(wW_ref[...], fw[0:EMBED, :],
                          preferred_element_type=jnp.float32)
    tK_ref[...] = jnp.dot(wK_ref[...], fw[EMBED:, :],
                          preferred_element_type=jnp.float32) + fcb_ref[...]


_tables = pl.pallas_call(
    _tables_body,
    out_shape=(
        jax.ShapeDtypeStruct((1000, EMBED), jnp.float32),
        jax.ShapeDtypeStruct((1000, EMBED), jnp.float32),
    ),
)


@functools.partial(
    pl.kernel,
    mesh=plsc.VectorSubcoreMesh(core_axis_name="c", subcore_axis_name="s"),
    compiler_params=pltpu.CompilerParams(use_tc_tiling_on_sc=False,
                                         needs_layout_passes=False),
    out_type=jax.ShapeDtypeStruct((_L * 8, _NW, _TILE), jnp.float32),
    scratch_types=[
        pltpu.VMEM((_CL, _BSLAB), jnp.int32),
        pltpu.VMEM((_CL, _BSLAB), jnp.int32),
        pltpu.VMEM((_CL * _BSLAB, OUT), jnp.float32),
        pltpu.VMEM((_CL * _BSLAB, OUT), jnp.float32),
        pltpu.VMEM((OUT * _BSLAB,), jnp.float32),
        pltpu.SemaphoreType.DMA,
        pltpu.SemaphoreType.DMA,
        pltpu.SemaphoreType.DMA,
    ],
)
def _sc_gather_add(tW_hbm, tK_hbm, wthr_hbm, week_hbm, out_hbm,
                   idx_a, idx_b, rows_a, rows_b, stag, sem_a, sem_b, sem_o):
    wid = lax.axis_index("s") * _NC + lax.axis_index("c")
    col = wid * _BSLAB
    v_d128 = lax.iota(jnp.int32, LANES) * _BSLAB

    def chunk_body(ci, carry):
        l0 = ci * _CL
        pltpu.sync_copy(wthr_hbm.at[pl.ds(l0, _CL), pl.ds(col, _BSLAB)],
                        idx_a)
        pltpu.sync_copy(week_hbm.at[pl.ds(l0, _CL), pl.ds(col, _BSLAB)],
                        idx_b)
        copies = []
        for j in range(_CL):
            dst = pl.ds(j * _BSLAB, _BSLAB)
            copies.append(pltpu.async_copy(
                tW_hbm.at[idx_a.at[j]], rows_a.at[dst], sem_a))
            copies.append(pltpu.async_copy(
                tK_hbm.at[idx_b.at[j]], rows_b.at[dst], sem_b))
        for c in copies:
            c.wait()

        for j in range(_CL):
            def b_body(b, acc, j=j):
                r = j * _BSLAB + b
                for k in range(OUT // LANES):
                    sl = pl.ds(k * LANES, LANES)
                    s = rows_a[r, sl] + rows_b[r, sl]
                    idx = v_d128 + (k * LANES * _BSLAB + b)
                    plsc.store_scatter(stag, [idx], s)
                return acc

            lax.fori_loop(0, _BSLAB, b_body, 0)
            outs = []
            for dt in range(8):
                outs.append(pltpu.async_copy(
                    stag.at[pl.ds(dt * _TILE, _TILE)],
                    out_hbm.at[(l0 + j) * 8 + dt, wid], sem_o))
            for c in outs:
                c.wait()
        return carry

    lax.fori_loop(0, _NCHUNKS, chunk_body, 0)


def kernel(weather, week, W_weather, W_week, fc_W, fc_b):
    tW, tK = _tables(W_weather, W_week, fc_W, fc_b.reshape(1, OUT))
    wthr_t = weather.astype(jnp.int32).T
    week_t = week.astype(jnp.int32).T
    o = _sc_gather_add(tW, tK, wthr_t, week_t)
    o = o.reshape(_L, 8, _NW, 8, 128)
    return o.transpose(2, 4, 0, 1, 3).reshape(_B, _L, OUT)


# speedup vs baseline: 1.4441x; 1.4441x over previous
import functools

import jax
import jax.numpy as jnp
from jax import lax
from jax.experimental import pallas as pl
from jax.experimental.pallas import tpu as pltpu
from jax.experimental.pallas import tpu_sc as plsc

EMBED = 64
OUT = 64
LANES = 16

_NC = 2
_NS = 16
_NW = _NC * _NS

_B = 4096
_L = 200
_BSLAB = _B // _NW
_CL = 2
_NCHUNKS = _L // _CL
_TILE = 8 * 128
_PITCH = 129


def _tables_body(wW_ref, wK_ref, fcW_ref, fcb_ref, tW_ref, tK_ref):
    fw = fcW_ref[...]
    tW_ref[...] = jnp.dot(wW_ref[...], fw[0:EMBED, :],
                          preferred_element_type=jnp.float32)
    tK_ref[...] = jnp.dot(wK_ref[...], fw[EMBED:, :],
                          preferred_element_type=jnp.float32) + fcb_ref[...]


_tables = pl.pallas_call(
    _tables_body,
    out_shape=(
        jax.ShapeDtypeStruct((1000, EMBED), jnp.float32),
        jax.ShapeDtypeStruct((1000, EMBED), jnp.float32),
    ),
)


@functools.partial(
    pl.kernel,
    mesh=plsc.VectorSubcoreMesh(core_axis_name="c", subcore_axis_name="s"),
    compiler_params=pltpu.CompilerParams(use_tc_tiling_on_sc=False,
                                         needs_layout_passes=False),
    out_type=jax.ShapeDtypeStruct((_L * 8, _NW, _TILE), jnp.float32),
    scratch_types=[
        pltpu.VMEM((_CL, _BSLAB), jnp.int32),
        pltpu.VMEM((_CL, _BSLAB), jnp.int32),
        pltpu.VMEM((_CL, _BSLAB), jnp.int32),
        pltpu.VMEM((_CL, _BSLAB), jnp.int32),
        pltpu.VMEM((_CL * _BSLAB, OUT), jnp.float32),
        pltpu.VMEM((_CL * _BSLAB, OUT), jnp.float32),
        pltpu.VMEM((_CL * _BSLAB, OUT), jnp.float32),
        pltpu.VMEM((_CL * _BSLAB, OUT), jnp.float32),
        pltpu.VMEM((OUT * _PITCH,), jnp.float32),
        pltpu.VMEM((8 * _TILE,), jnp.float32),
        pltpu.VMEM((8 * _TILE,), jnp.float32),
        pltpu.SemaphoreType.DMA,
        pltpu.SemaphoreType.DMA,
        pltpu.SemaphoreType.DMA,
        pltpu.SemaphoreType.DMA,
    ],
)
def _sc_gather_add(tW_hbm, tK_hbm, wthr_hbm, week_hbm, out_hbm,
                   ia0, ia1, ib0, ib1, ra0, ra1, rb0, rb1,
                   pad, pk0, pk1, sg0, sg1, so0, so1):
    wid = lax.axis_index("s") * _NC + lax.axis_index("c")
    col = wid * _BSLAB
    v_d129 = lax.iota(jnp.int32, LANES) * _PITCH

    idx_a = (ia0, ia1)
    idx_b = (ib0, ib1)
    rows_a = (ra0, ra1)
    rows_b = (rb0, rb1)
    packs = (pk0, pk1)
    sg = (sg0, sg1)
    so = (so0, so1)

    def fire(c, slot):
        l0 = c * _CL
        pltpu.sync_copy(wthr_hbm.at[pl.ds(l0, _CL), pl.ds(col, _BSLAB)],
                        idx_a[slot])
        pltpu.sync_copy(week_hbm.at[pl.ds(l0, _CL), pl.ds(col, _BSLAB)],
                        idx_b[slot])
        for j in range(_CL):
            dst = pl.ds(j * _BSLAB, _BSLAB)
            pltpu.async_copy(tW_hbm.at[idx_a[slot].at[j]],
                             rows_a[slot].at[dst], sg[slot])
            pltpu.async_copy(tK_hbm.at[idx_b[slot].at[j]],
                             rows_b[slot].at[dst], sg[slot])

    def wait_gathers(slot):
        for j in range(_CL):
            dst = pl.ds(j * _BSLAB, _BSLAB)
            pltpu.make_async_copy(tW_hbm.at[idx_a[slot].at[j]],
                                  rows_a[slot].at[dst], sg[slot]).wait()
            pltpu.make_async_copy(tK_hbm.at[idx_b[slot].at[j]],
                                  rows_b[slot].at[dst], sg[slot]).wait()

    fire(0, 0)

    def pair_body(p, carry):
        for s in range(2):
            c = p * 2 + s
            slot = s
            wait_gathers(slot)

            @pl.when(c + 1 < _NCHUNKS)
            def _():
                fire(c + 1, 1 - slot)

            for j in range(_CL):
                ra = rows_a[slot]
                rb = rows_b[slot]
                pk = packs[j]

                def b_body(i, acc, ra=ra, rb=rb, j=j):
                    for u in range(2):
                        b = i * 2 + u
                        r = j * _BSLAB + b
                        for k in range(OUT // LANES):
                            sl = pl.ds(k * LANES, LANES)
                            s_v = ra[r, sl] + rb[r, sl]
                            idx = v_d129 + (k * LANES * _PITCH + b)
                            plsc.store_scatter(pad, [idx], s_v)
                    return acc

                lax.fori_loop(0, _BSLAB // 2, b_body, 0)

                @pl.when(c > 0)
                def _(pk=pk, j=j):
                    for dt in range(8):
                        pltpu.make_async_copy(
                            pk.at[pl.ds(dt * _TILE, _TILE)],
                            out_hbm.at[dt, wid], so[j]).wait()

                def d_body(i, acc, pk=pk):
                    for u in range(2):
                        d = i * 2 + u
                        for m in range(8):
                            pk[pl.ds(d * 128 + m * LANES, LANES)] = (
                                pad[pl.ds(d * _PITCH + m * LANES, LANES)])
                    return acc

                lax.fori_loop(0, OUT // 2, d_body, 0)

                g0 = (c * _CL + j) * 8
                for dt in range(8):
                    pltpu.async_copy(pk.at[pl.ds(dt * _TILE, _TILE)],
                                     out_hbm.at[g0 + dt, wid], so[j])
        return carry

    lax.fori_loop(0, _NCHUNKS // 2, pair_body, 0)

    for j in range(_CL):
        for dt in range(8):
            pltpu.make_async_copy(packs[j].at[pl.ds(dt * _TILE, _TILE)],
                                  out_hbm.at[dt, wid], so[j]).wait()


def kernel(weather, week, W_weather, W_week, fc_W, fc_b):
    tW, tK = _tables(W_weather, W_week, fc_W, fc_b.reshape(1, OUT))
    wthr_t = weather.astype(jnp.int32).T
    week_t = week.astype(jnp.int32).T
    o = _sc_gather_add(tW, tK, wthr_t, week_t)
    o = o.reshape(_L, 8, _NW, 8, 128)
    return o.transpose(2, 4, 0, 1, 3).reshape(_B, _L, OUT)


# speedup vs baseline: 3.4641x; 2.3988x over previous
import functools

import jax
import jax.numpy as jnp
from jax import lax
from jax.experimental import pallas as pl
from jax.experimental.pallas import tpu as pltpu
from jax.experimental.pallas import tpu_sc as plsc

EMBED = 64
OUT = 64
LANES = 16

_NC = 2
_NS = 16
_NW = _NC * _NS

_B = 4096
_L = 200
_BSLAB = _B // _NW
_CL = 2
_NCHUNKS = _L // _CL
_TILE = 8 * 128
_PITCH = 129


def _tables_body(wW_ref, wK_ref, fcW_ref, fcb_ref, tW_ref, tK_ref):
    fw = fcW_ref[...]
    tW_ref[...] = jnp.dot(wW_ref[...], fw[0:EMBED, :],
                          preferred_element_type=jnp.float32)
    tK_ref[...] = jnp.dot(wK_ref[...], fw[EMBED:, :],
                          preferred_element_type=jnp.float32) + fcb_ref[...]


_tables = pl.pallas_call(
    _tables_body,
    out_shape=(
        jax.ShapeDtypeStruct((1000, EMBED), jnp.float32),
        jax.ShapeDtypeStruct((1000, EMBED), jnp.float32),
    ),
)


@functools.partial(
    pl.kernel,
    mesh=plsc.VectorSubcoreMesh(core_axis_name="c", subcore_axis_name="s"),
    compiler_params=pltpu.CompilerParams(use_tc_tiling_on_sc=False,
                                         needs_layout_passes=False),
    out_type=jax.ShapeDtypeStruct((_L * 8, _NW, _TILE), jnp.float32),
    scratch_types=[
        pltpu.VMEM((_CL, _BSLAB), jnp.int32),
        pltpu.VMEM((_CL, _BSLAB), jnp.int32),
        pltpu.VMEM((_CL, _BSLAB), jnp.int32),
        pltpu.VMEM((_CL, _BSLAB), jnp.int32),
        pltpu.VMEM((_CL * _BSLAB, OUT), jnp.float32),
        pltpu.VMEM((_CL * _BSLAB, OUT), jnp.float32),
        pltpu.VMEM((_CL * _BSLAB, OUT), jnp.float32),
        pltpu.VMEM((_CL * _BSLAB, OUT), jnp.float32),
        pltpu.VMEM((OUT * _PITCH,), jnp.float32),
        pltpu.VMEM((8 * _TILE,), jnp.float32),
        pltpu.VMEM((8 * _TILE,), jnp.float32),
        pltpu.SemaphoreType.DMA,
        pltpu.SemaphoreType.DMA,
        pltpu.SemaphoreType.DMA,
        pltpu.SemaphoreType.DMA,
    ],
)
def _sc_gather_add(tW_hbm, tK_hbm, wthr_hbm, week_hbm, out_hbm,
                   ia0, ia1, ib0, ib1, ra0, ra1, rb0, rb1,
                   pad, pk0, pk1, sg0, sg1, so0, so1):
    wid = lax.axis_index("s") * _NC + lax.axis_index("c")
    col = wid * _BSLAB
    v_d129 = lax.iota(jnp.int32, LANES) * _PITCH

    idx_a = (ia0, ia1)
    idx_b = (ib0, ib1)
    rows_a = (ra0, ra1)
    rows_b = (rb0, rb1)
    packs = (pk0, pk1)
    sg = (sg0, sg1)
    so = (so0, so1)

    def fire(c, slot):
        l0 = c * _CL
        pltpu.sync_copy(wthr_hbm.at[pl.ds(l0, _CL), pl.ds(col, _BSLAB)],
                        idx_a[slot])
        pltpu.sync_copy(week_hbm.at[pl.ds(l0, _CL), pl.ds(col, _BSLAB)],
                        idx_b[slot])
        for j in range(_CL):
            dst = pl.ds(j * _BSLAB, _BSLAB)
            pltpu.async_copy(tW_hbm.at[idx_a[slot].at[j]],
                             rows_a[slot].at[dst], sg[slot])
            pltpu.async_copy(tK_hbm.at[idx_b[slot].at[j]],
                             rows_b[slot].at[dst], sg[slot])

    def wait_gathers(slot):
        for j in range(_CL):
            dst = pl.ds(j * _BSLAB, _BSLAB)
            pltpu.make_async_copy(tW_hbm.at[idx_a[slot].at[j]],
                                  rows_a[slot].at[dst], sg[slot]).wait()
            pltpu.make_async_copy(tK_hbm.at[idx_b[slot].at[j]],
                                  rows_b[slot].at[dst], sg[slot]).wait()

    fire(0, 0)

    def pair_body(p, carry):
        for s in range(2):
            c = p * 2 + s
            slot = s
            wait_gathers(slot)

            @pl.when(c + 1 < _NCHUNKS)
            def _():
                fire(c + 1, 1 - slot)

            for j in range(_CL):
                ra = rows_a[slot]
                rb = rows_b[slot]
                pk = packs[j]

                @plsc.parallel_loop(0, _BSLAB, 1, unroll=4)
                def _(b, ra=ra, rb=rb, j=j):
                    r = j * _BSLAB + b
                    for k in range(OUT // LANES):
                        sl = pl.ds(k * LANES, LANES)
                        s_v = ra[r, sl] + rb[r, sl]
                        idx = v_d129 + (k * LANES * _PITCH + b)
                        plsc.store_scatter(pad, [idx], s_v)

                @pl.when(c > 0)
                def _(pk=pk, j=j):
                    for dt in range(8):
                        pltpu.make_async_copy(
                            pk.at[pl.ds(dt * _TILE, _TILE)],
                            out_hbm.at[dt, wid], so[j]).wait()

                @plsc.parallel_loop(0, OUT, 1, unroll=4)
                def _(d, pk=pk):
                    for m in range(8):
                        pk[pl.ds(d * 128 + m * LANES, LANES)] = (
                            pad[pl.ds(d * _PITCH + m * LANES, LANES)])

                g0 = (c * _CL + j) * 8
                for dt in range(8):
                    pltpu.async_copy(pk.at[pl.ds(dt * _TILE, _TILE)],
                                     out_hbm.at[g0 + dt, wid], so[j])
        return carry

    lax.fori_loop(0, _NCHUNKS // 2, pair_body, 0)

    for j in range(_CL):
        for dt in range(8):
            pltpu.make_async_copy(packs[j].at[pl.ds(dt * _TILE, _TILE)],
                                  out_hbm.at[dt, wid], so[j]).wait()


def kernel(weather, week, W_weather, W_week, fc_W, fc_b):
    tW, tK = _tables(W_weather, W_week, fc_W, fc_b.reshape(1, OUT))
    wthr_t = weather.astype(jnp.int32).T
    week_t = week.astype(jnp.int32).T
    o = _sc_gather_add(tW, tK, wthr_t, week_t)
    o = o.reshape(_L, 8, _NW, 8, 128)
    return o.transpose(2, 4, 0, 1, 3).reshape(_B, _L, OUT)


# speedup vs baseline: 4.4400x; 1.2817x over previous
import functools

import jax
import jax.numpy as jnp
from jax import lax
from jax.experimental import pallas as pl
from jax.experimental.pallas import tpu as pltpu
from jax.experimental.pallas import tpu_sc as plsc

EMBED = 64
OUT = 64
LANES = 16

_NC = 2
_NS = 16
_NW = _NC * _NS

_B = 4096
_L = 200
_BSLAB = _B // _NW
_CL = 2
_NCHUNKS = _L // _CL
_TILE = 8 * 128
_PITCH = 129


def _tables_body(wW_ref, wK_ref, fcW_ref, fcb_ref, tW_ref, tK_ref):
    fw = fcW_ref[...]
    tW_ref[...] = jnp.dot(wW_ref[...], fw[0:EMBED, :],
                          preferred_element_type=jnp.float32)
    tK_ref[...] = jnp.dot(wK_ref[...], fw[EMBED:, :],
                          preferred_element_type=jnp.float32) + fcb_ref[...]


_tables = pl.pallas_call(
    _tables_body,
    out_shape=(
        jax.ShapeDtypeStruct((1000, EMBED), jnp.float32),
        jax.ShapeDtypeStruct((1000, EMBED), jnp.float32),
    ),
)


@functools.partial(
    pl.kernel,
    mesh=plsc.VectorSubcoreMesh(core_axis_name="c", subcore_axis_name="s"),
    compiler_params=pltpu.CompilerParams(use_tc_tiling_on_sc=False,
                                         needs_layout_passes=False),
    out_type=jax.ShapeDtypeStruct((_L * 8, _NW, _TILE), jnp.float32),
    scratch_types=[
        pltpu.VMEM((_CL, _BSLAB), jnp.int32),
        pltpu.VMEM((_CL, _BSLAB), jnp.int32),
        pltpu.VMEM((_CL, _BSLAB), jnp.int32),
        pltpu.VMEM((_CL, _BSLAB), jnp.int32),
        pltpu.VMEM((_CL * _BSLAB, OUT), jnp.float32),
        pltpu.VMEM((_CL * _BSLAB, OUT), jnp.float32),
        pltpu.VMEM((_CL * _BSLAB, OUT), jnp.float32),
        pltpu.VMEM((_CL * _BSLAB, OUT), jnp.float32),
        pltpu.VMEM((OUT * _PITCH,), jnp.float32),
        pltpu.VMEM((8 * _TILE,), jnp.float32),
        pltpu.VMEM((8 * _TILE,), jnp.float32),
        pltpu.VMEM_SHARED((1000, EMBED), jnp.float32),
        pltpu.VMEM_SHARED((1000, EMBED), jnp.float32),
        pltpu.SemaphoreType.DMA,
        pltpu.SemaphoreType.DMA,
        pltpu.SemaphoreType.DMA,
        pltpu.SemaphoreType.DMA,
    ],
)
def _sc_gather_add(tW_hbm, tK_hbm, wthr_hbm, week_hbm, out_hbm,
                   ia0, ia1, ib0, ib1, ra0, ra1, rb0, rb1,
                   pad, pk0, pk1, shW, shK, sg0, sg1, so0, so1):
    wid = lax.axis_index("s") * _NC + lax.axis_index("c")
    col = wid * _BSLAB
    v_d129 = lax.iota(jnp.int32, LANES) * _PITCH

    @pl.when(lax.axis_index("s") == 0)
    def _():
        pltpu.sync_copy(tW_hbm, shW)
        pltpu.sync_copy(tK_hbm, shK)
    plsc.subcore_barrier()

    idx_a = (ia0, ia1)
    idx_b = (ib0, ib1)
    rows_a = (ra0, ra1)
    rows_b = (rb0, rb1)
    packs = (pk0, pk1)
    sg = (sg0, sg1)
    so = (so0, so1)

    def fire(c, slot):
        l0 = c * _CL
        pltpu.sync_copy(wthr_hbm.at[pl.ds(l0, _CL), pl.ds(col, _BSLAB)],
                        idx_a[slot])
        pltpu.sync_copy(week_hbm.at[pl.ds(l0, _CL), pl.ds(col, _BSLAB)],
                        idx_b[slot])
        for j in range(_CL):
            dst = pl.ds(j * _BSLAB, _BSLAB)
            pltpu.async_copy(shW.at[idx_a[slot].at[j]],
                             rows_a[slot].at[dst], sg[slot])
            pltpu.async_copy(shK.at[idx_b[slot].at[j]],
                             rows_b[slot].at[dst], sg[slot])

    def wait_gathers(slot):
        for j in range(_CL):
            dst = pl.ds(j * _BSLAB, _BSLAB)
            pltpu.make_async_copy(shW.at[idx_a[slot].at[j]],
                                  rows_a[slot].at[dst], sg[slot]).wait()
            pltpu.make_async_copy(shK.at[idx_b[slot].at[j]],
                                  rows_b[slot].at[dst], sg[slot]).wait()

    fire(0, 0)

    def pair_body(p, carry):
        for s in range(2):
            c = p * 2 + s
            slot = s
            wait_gathers(slot)

            @pl.when(c + 1 < _NCHUNKS)
            def _():
                fire(c + 1, 1 - slot)

            for j in range(_CL):
                ra = rows_a[slot]
                rb = rows_b[slot]
                pk = packs[j]

                @plsc.parallel_loop(0, _BSLAB, 1, unroll=4)
                def _(b, ra=ra, rb=rb, j=j):
                    r = j * _BSLAB + b
                    for k in range(OUT // LANES):
                        sl = pl.ds(k * LANES, LANES)
                        s_v = ra[r, sl] + rb[r, sl]
                        idx = v_d129 + (k * LANES * _PITCH + b)
                        plsc.store_scatter(pad, [idx], s_v)

                @pl.when(c > 0)
                def _(pk=pk, j=j):
                    for dt in range(8):
                        pltpu.make_async_copy(
                            pk.at[pl.ds(dt * _TILE, _TILE)],
                            out_hbm.at[dt, wid], so[j]).wait()

                @plsc.parallel_loop(0, OUT, 1, unroll=4)
                def _(d, pk=pk):
                    for m in range(8):
                        pk[pl.ds(d * 128 + m * LANES, LANES)] = (
                            pad[pl.ds(d * _PITCH + m * LANES, LANES)])

                g0 = (c * _CL + j) * 8
                for dt in range(8):
                    pltpu.async_copy(pk.at[pl.ds(dt * _TILE, _TILE)],
                                     out_hbm.at[g0 + dt, wid], so[j])
        return carry

    lax.fori_loop(0, _NCHUNKS // 2, pair_body, 0)

    for j in range(_CL):
        for dt in range(8):
            pltpu.make_async_copy(packs[j].at[pl.ds(dt * _TILE, _TILE)],
                                  out_hbm.at[dt, wid], so[j]).wait()


def kernel(weather, week, W_weather, W_week, fc_W, fc_b):
    tW, tK = _tables(W_weather, W_week, fc_W, fc_b.reshape(1, OUT))
    wthr_t = weather.astype(jnp.int32).T
    week_t = week.astype(jnp.int32).T
    o = _sc_gather_add(tW, tK, wthr_t, week_t)
    o = o.reshape(_L, 8, _NW, 8, 128)
    return o.transpose(2, 4, 0, 1, 3).reshape(_B, _L, OUT)


# speedup vs baseline: 5.7515x; 1.2954x over previous
import functools

import jax
import jax.numpy as jnp
from jax import lax
from jax.experimental import pallas as pl
from jax.experimental.pallas import tpu as pltpu
from jax.experimental.pallas import tpu_sc as plsc

EMBED = 64
OUT = 64
LANES = 16

_NC = 2
_NS = 16
_NW = _NC * _NS

_B = 4096
_L = 200
_BSLAB = _B // _NW
_CL = 2
_NCHUNKS = _L // _CL
_TILE = 8 * 128
_PITCH = 129


def _tables_body(wW_ref, wK_ref, fcW_ref, fcb_ref, tW_ref, tK_ref):
    fw = fcW_ref[...]
    tW_ref[...] = jnp.dot(wW_ref[...], fw[0:EMBED, :],
                          preferred_element_type=jnp.float32)
    tK_ref[...] = jnp.dot(wK_ref[...], fw[EMBED:, :],
                          preferred_element_type=jnp.float32) + fcb_ref[...]


_tables = pl.pallas_call(
    _tables_body,
    out_shape=(
        jax.ShapeDtypeStruct((1000, EMBED), jnp.float32),
        jax.ShapeDtypeStruct((1000, EMBED), jnp.float32),
    ),
)


@functools.partial(
    pl.kernel,
    mesh=plsc.VectorSubcoreMesh(core_axis_name="c", subcore_axis_name="s"),
    compiler_params=pltpu.CompilerParams(use_tc_tiling_on_sc=False,
                                         needs_layout_passes=False),
    out_type=jax.ShapeDtypeStruct((_L * 8, _NW, _TILE), jnp.float32),
    scratch_types=[
        pltpu.VMEM((_CL, _BSLAB), jnp.int32),
        pltpu.VMEM((_CL, _BSLAB), jnp.int32),
        pltpu.VMEM((_CL, _BSLAB), jnp.int32),
        pltpu.VMEM((_CL, _BSLAB), jnp.int32),
        pltpu.VMEM((_CL * _BSLAB, OUT), jnp.float32),
        pltpu.VMEM((_CL * _BSLAB, OUT), jnp.float32),
        pltpu.VMEM((_CL * _BSLAB, OUT), jnp.float32),
        pltpu.VMEM((_CL * _BSLAB, OUT), jnp.float32),
        pltpu.VMEM((OUT * _PITCH,), jnp.float32),
        pltpu.VMEM((8 * _TILE,), jnp.float32),
        pltpu.VMEM((8 * _TILE,), jnp.float32),
        pltpu.VMEM_SHARED((1000, EMBED), jnp.float32),
        pltpu.VMEM_SHARED((1000, EMBED), jnp.float32),
        pltpu.SemaphoreType.DMA,
        pltpu.SemaphoreType.DMA,
        pltpu.SemaphoreType.DMA,
        pltpu.SemaphoreType.DMA,
        pltpu.SemaphoreType.DMA,
        pltpu.SemaphoreType.DMA,
    ],
)
def _sc_gather_add(tW_hbm, tK_hbm, wthr_hbm, week_hbm, out_hbm,
                   ia0, ia1, ib0, ib1, ra0, ra1, rb0, rb1,
                   pad, pk0, pk1, shW, shK, sg0, sg1, so0, so1, si0, si1):
    wid = lax.axis_index("s") * _NC + lax.axis_index("c")
    col = wid * _BSLAB
    v_d129 = lax.iota(jnp.int32, LANES) * _PITCH

    @pl.when(lax.axis_index("s") == 0)
    def _():
        pltpu.sync_copy(tW_hbm, shW)
        pltpu.sync_copy(tK_hbm, shK)
    plsc.subcore_barrier()

    idx_a = (ia0, ia1)
    idx_b = (ib0, ib1)
    rows_a = (ra0, ra1)
    rows_b = (rb0, rb1)
    packs = (pk0, pk1)
    sg = (sg0, sg1)
    so = (so0, so1)
    si = (si0, si1)

    def fire_idx(c, slot):
        l0 = c * _CL
        pltpu.async_copy(wthr_hbm.at[pl.ds(l0, _CL), pl.ds(col, _BSLAB)],
                         idx_a[slot], si[slot])
        pltpu.async_copy(week_hbm.at[pl.ds(l0, _CL), pl.ds(col, _BSLAB)],
                         idx_b[slot], si[slot])

    def wait_idx(c, slot):
        l0 = c * _CL
        pltpu.make_async_copy(
            wthr_hbm.at[pl.ds(l0, _CL), pl.ds(col, _BSLAB)],
            idx_a[slot], si[slot]).wait()
        pltpu.make_async_copy(
            week_hbm.at[pl.ds(l0, _CL), pl.ds(col, _BSLAB)],
            idx_b[slot], si[slot]).wait()

    def fire_gathers(slot):
        for j in range(_CL):
            dst = pl.ds(j * _BSLAB, _BSLAB)
            pltpu.async_copy(shW.at[idx_a[slot].at[j]],
                             rows_a[slot].at[dst], sg[slot])
            pltpu.async_copy(shK.at[idx_b[slot].at[j]],
                             rows_b[slot].at[dst], sg[slot])

    def wait_gathers(slot):
        for j in range(_CL):
            dst = pl.ds(j * _BSLAB, _BSLAB)
            pltpu.make_async_copy(shW.at[idx_a[slot].at[j]],
                                  rows_a[slot].at[dst], sg[slot]).wait()
            pltpu.make_async_copy(shK.at[idx_b[slot].at[j]],
                                  rows_b[slot].at[dst], sg[slot]).wait()

    fire_idx(0, 0)
    fire_idx(1, 1)
    wait_idx(0, 0)
    fire_gathers(0)

    def pair_body(p, carry):
        for s in range(2):
            c = p * 2 + s
            slot = s
            wait_gathers(slot)

            @pl.when(c + 1 < _NCHUNKS)
            def _():
                wait_idx(c + 1, 1 - slot)
                fire_gathers(1 - slot)

            @pl.when(c + 2 < _NCHUNKS)
            def _():
                fire_idx(c + 2, slot)

            for j in range(_CL):
                ra = rows_a[slot]
                rb = rows_b[slot]
                pk = packs[j]

                @plsc.parallel_loop(0, _BSLAB, 1, unroll=4)
                def _(b, ra=ra, rb=rb, j=j):
                    r = j * _BSLAB + b
                    for k in range(OUT // LANES):
                        sl = pl.ds(k * LANES, LANES)
                        s_v = ra[r, sl] + rb[r, sl]
                        idx = v_d129 + (k * LANES * _PITCH + b)
                        plsc.store_scatter(pad, [idx], s_v)

                @pl.when(c > 0)
                def _(pk=pk, j=j):
                    for dt in range(8):
                        pltpu.make_async_copy(
                            pk.at[pl.ds(dt * _TILE, _TILE)],
                            out_hbm.at[dt, wid], so[j]).wait()

                @plsc.parallel_loop(0, OUT, 1, unroll=4)
                def _(d, pk=pk):
                    for m in range(8):
                        pk[pl.ds(d * 128 + m * LANES, LANES)] = (
                            pad[pl.ds(d * _PITCH + m * LANES, LANES)])

                g0 = (c * _CL + j) * 8
                for dt in range(8):
                    pltpu.async_copy(pk.at[pl.ds(dt * _TILE, _TILE)],
                                     out_hbm.at[g0 + dt, wid], so[j])
        return carry

    lax.fori_loop(0, _NCHUNKS // 2, pair_body, 0)

    for j in range(_CL):
        for dt in range(8):
            pltpu.make_async_copy(packs[j].at[pl.ds(dt * _TILE, _TILE)],
                                  out_hbm.at[dt, wid], so[j]).wait()


def kernel(weather, week, W_weather, W_week, fc_W, fc_b):
    tW, tK = _tables(W_weather, W_week, fc_W, fc_b.reshape(1, OUT))
    wthr_t = weather.astype(jnp.int32).T
    week_t = week.astype(jnp.int32).T
    o = _sc_gather_add(tW, tK, wthr_t, week_t)
    o = o.reshape(_L, 8, _NW, 8, 128)
    return o.transpose(2, 4, 0, 1, 3).reshape(_B, _L, OUT)


# speedup vs baseline: 5.8261x; 1.0130x over previous
import functools

import jax
import jax.numpy as jnp
from jax import lax
from jax.experimental import pallas as pl
from jax.experimental.pallas import tpu as pltpu
from jax.experimental.pallas import tpu_sc as plsc

EMBED = 64
OUT = 64
LANES = 16

_NC = 2
_NS = 16
_NW = _NC * _NS

_B = 4096
_L = 200
_BSLAB = _B // _NW
_NSLOT = 4
_NCHUNKS = _L
_TILE = 8 * 128
_PITCH = 129


def _tables_body(wW_ref, wK_ref, fcW_ref, fcb_ref, tW_ref, tK_ref):
    fw = fcW_ref[...]
    tW_ref[...] = jnp.dot(wW_ref[...], fw[0:EMBED, :],
                          preferred_element_type=jnp.float32)
    tK_ref[...] = jnp.dot(wK_ref[...], fw[EMBED:, :],
                          preferred_element_type=jnp.float32) + fcb_ref[...]


_tables = pl.pallas_call(
    _tables_body,
    out_shape=(
        jax.ShapeDtypeStruct((1000, EMBED), jnp.float32),
        jax.ShapeDtypeStruct((1000, EMBED), jnp.float32),
    ),
)


@functools.partial(
    pl.kernel,
    mesh=plsc.VectorSubcoreMesh(core_axis_name="c", subcore_axis_name="s"),
    compiler_params=pltpu.CompilerParams(use_tc_tiling_on_sc=False,
                                         needs_layout_passes=False),
    out_type=jax.ShapeDtypeStruct((_L * 8, _NW, _TILE), jnp.float32),
    scratch_types=(
        [pltpu.VMEM((1, _BSLAB), jnp.int32)] * _NSLOT
        + [pltpu.VMEM((1, _BSLAB), jnp.int32)] * _NSLOT
        + [pltpu.VMEM((_BSLAB, OUT), jnp.float32)] * _NSLOT
        + [pltpu.VMEM((_BSLAB, OUT), jnp.float32)] * _NSLOT
        + [
            pltpu.VMEM((OUT * _PITCH,), jnp.float32),
            pltpu.VMEM((8 * _TILE,), jnp.float32),
            pltpu.VMEM((8 * _TILE,), jnp.float32),
            pltpu.VMEM_SHARED((1000, EMBED), jnp.float32),
            pltpu.VMEM_SHARED((1000, EMBED), jnp.float32),
        ]
        + [pltpu.SemaphoreType.DMA] * _NSLOT
        + [pltpu.SemaphoreType.DMA] * 2
        + [pltpu.SemaphoreType.DMA] * _NSLOT
    ),
)
def _sc_gather_add(tW_hbm, tK_hbm, wthr_hbm, week_hbm, out_hbm,
                   ia0, ia1, ia2, ia3, ib0, ib1, ib2, ib3,
                   ra0, ra1, ra2, ra3, rb0, rb1, rb2, rb3,
                   pad, pk0, pk1, shW, shK,
                   sg0, sg1, sg2, sg3, so0, so1, si0, si1, si2, si3):
    wid = lax.axis_index("s") * _NC + lax.axis_index("c")
    col = wid * _BSLAB
    v_d129 = lax.iota(jnp.int32, LANES) * _PITCH

    @pl.when(lax.axis_index("s") == 0)
    def _():
        pltpu.sync_copy(tW_hbm, shW)
        pltpu.sync_copy(tK_hbm, shK)
    plsc.subcore_barrier()

    idx_a = (ia0, ia1, ia2, ia3)
    idx_b = (ib0, ib1, ib2, ib3)
    rows_a = (ra0, ra1, ra2, ra3)
    rows_b = (rb0, rb1, rb2, rb3)
    packs = (pk0, pk1)
    sg = (sg0, sg1, sg2, sg3)
    so = (so0, so1)
    si = (si0, si1, si2, si3)

    def fire_idx(c, slot):
        pltpu.async_copy(wthr_hbm.at[pl.ds(c, 1), pl.ds(col, _BSLAB)],
                         idx_a[slot], si[slot])
        pltpu.async_copy(week_hbm.at[pl.ds(c, 1), pl.ds(col, _BSLAB)],
                         idx_b[slot], si[slot])

    def wait_idx(c, slot):
        pltpu.make_async_copy(wthr_hbm.at[pl.ds(c, 1), pl.ds(col, _BSLAB)],
                              idx_a[slot], si[slot]).wait()
        pltpu.make_async_copy(week_hbm.at[pl.ds(c, 1), pl.ds(col, _BSLAB)],
                              idx_b[slot], si[slot]).wait()

    def fire_gathers(slot):
        pltpu.async_copy(shW.at[idx_a[slot].at[0]], rows_a[slot], sg[slot])
        pltpu.async_copy(shK.at[idx_b[slot].at[0]], rows_b[slot], sg[slot])

    def wait_gathers(slot):
        pltpu.make_async_copy(shW.at[idx_a[slot].at[0]],
                              rows_a[slot], sg[slot]).wait()
        pltpu.make_async_copy(shK.at[idx_b[slot].at[0]],
                              rows_b[slot], sg[slot]).wait()

    for c0 in range(_NSLOT):
        fire_idx(c0, c0)
    for c0 in range(_NSLOT - 1):
        wait_idx(c0, c0)
        fire_gathers(c0)

    def quad_body(q, carry):
        for s in range(_NSLOT):
            c = q * _NSLOT + s
            slot = s
            wait_gathers(slot)

            @pl.when(c + _NSLOT - 1 < _NCHUNKS)
            def _():
                wait_idx(c + _NSLOT - 1, (s + _NSLOT - 1) % _NSLOT)
                fire_gathers((s + _NSLOT - 1) % _NSLOT)

            @pl.when(c + _NSLOT < _NCHUNKS)
            def _():
                fire_idx(c + _NSLOT, slot)

            ra = rows_a[slot]
            rb = rows_b[slot]
            pk = packs[s % 2]
            so_j = so[s % 2]

            @plsc.parallel_loop(0, _BSLAB, 1, unroll=4)
            def _(b, ra=ra, rb=rb):
                for k in range(OUT // LANES):
                    sl = pl.ds(k * LANES, LANES)
                    s_v = ra[b, sl] + rb[b, sl]
                    idx = v_d129 + (k * LANES * _PITCH + b)
                    plsc.store_scatter(pad, [idx], s_v)

            @pl.when(c > 1)
            def _(pk=pk, so_j=so_j):
                for dt in range(8):
                    pltpu.make_async_copy(
                        pk.at[pl.ds(dt * _TILE, _TILE)],
                        out_hbm.at[dt, wid], so_j).wait()

            @plsc.parallel_loop(0, OUT, 1, unroll=4)
            def _(d, pk=pk):
                for m in range(8):
                    pk[pl.ds(d * 128 + m * LANES, LANES)] = (
                        pad[pl.ds(d * _PITCH + m * LANES, LANES)])

            for dt in range(8):
                pltpu.async_copy(pk.at[pl.ds(dt * _TILE, _TILE)],
                                 out_hbm.at[c * 8 + dt, wid], so_j)
        return carry

    lax.fori_loop(0, _NCHUNKS // _NSLOT, quad_body, 0)

    for j in range(2):
        for dt in range(8):
            pltpu.make_async_copy(packs[j].at[pl.ds(dt * _TILE, _TILE)],
                                  out_hbm.at[dt, wid], so[j]).wait()


def kernel(weather, week, W_weather, W_week, fc_W, fc_b):
    tW, tK = _tables(W_weather, W_week, fc_W, fc_b.reshape(1, OUT))
    wthr_t = weather.astype(jnp.int32).T
    week_t = week.astype(jnp.int32).T
    o = _sc_gather_add(tW, tK, wthr_t, week_t)
    o = o.reshape(_L, 8, _NW, 8, 128)
    return o.transpose(2, 4, 0, 1, 3).reshape(_B, _L, OUT)
